# baseline (speedup 1.0000x reference)
import functools

import jax
import jax.numpy as jnp
from jax import lax
from jax.experimental import pallas as pl
from jax.experimental.pallas import tpu as pltpu
from jax.experimental.pallas import tpu_sc as plsc

_B, _C, _T = 8, 128, 5120
_NX, _NY = 256, 256
_DX, _DY = 0.001, 0.001
_VS, _DT = 1550.0, 5e-08
_TCLIP = 4655
_W = 1024
_NC, _NS = 2, 16
_BX, _BY = 64, 32
_L = 16


def _t2d_body(o_ref):
    a = lax.broadcasted_iota(jnp.int32, (_NX, _NY), 0).astype(jnp.float32)
    b = lax.broadcasted_iota(jnp.int32, (_NX, _NY), 1).astype(jnp.float32)
    dis = jnp.sqrt((a * _DX) ** 2 + (b * _DY) ** 2)
    t = (dis / _VS / _DT).astype(jnp.int32)
    o_ref[...] = jnp.clip(t, 0, _TCLIP)


def _t2d_table():
    return pl.pallas_call(
        _t2d_body,
        out_shape=jax.ShapeDtypeStruct((_NX, _NY), jnp.int32),
    )()


_mesh = plsc.VectorSubcoreMesh(core_axis_name="core", subcore_axis_name="sub")


@functools.partial(
    pl.kernel,
    out_type=jax.ShapeDtypeStruct((_B, _NX, _NY), jnp.float32),
    mesh=_mesh,
    compiler_params=pltpu.CompilerParams(
        use_tc_tiling_on_sc=False, needs_layout_passes=False),
    scratch_types=[
        pltpu.VMEM((_NX * _NY,), jnp.int32),
        pltpu.VMEM((_C + _L,), jnp.int32),
        pltpu.VMEM((_C + _L,), jnp.int32),
        pltpu.VMEM((_C + _L,), jnp.int32),
        pltpu.VMEM((_W * _B,), jnp.float32),
        pltpu.VMEM((_W * _B,), jnp.float32),
        pltpu.VMEM((_B, _BX, _BY), jnp.float32),
        pltpu.SemaphoreType.DMA,
        pltpu.SemaphoreType.DMA,
    ],
)
def _sc_das(sd_ref, t2d_ref, xs_ref, ys_ref, out_ref,
            t2d_v, xs_v, ys_v, basev, win0, win1, acc, sem0, sem1):
    cid = lax.axis_index("core")
    sid = lax.axis_index("sub")
    wid = sid * _NC + cid
    bx = (wid // 8) * _BX
    by = (wid % 8) * _BY

    pltpu.sync_copy(t2d_ref, t2d_v)
    pltpu.sync_copy(xs_ref, xs_v)
    pltpu.sync_copy(ys_ref, ys_v)

    def c_base(g, carry):
        xv = xs_v[pl.ds(g * _L, _L)]
        yv = ys_v[pl.ds(g * _L, _L)]
        amin = jnp.abs(jnp.clip(xv, bx, bx + _BX - 1) - xv)
        bmin = jnp.abs(jnp.clip(yv, by, by + _BY - 1) - yv)
        tmin = plsc.load_gather(t2d_v, [amin * 256 + bmin])
        basev[pl.ds(g * _L, _L)] = jnp.minimum(tmin >> 3, (_T - _W) // 8)
        return carry
    lax.fori_loop(0, _C // _L, c_base, 0)

    z = jnp.zeros((_L,), jnp.float32)
    def zero(k, carry):
        acc[k >> 7, (k >> 1) & 63, pl.ds((k & 1) * _L, _L)] = z
        return carry
    lax.fori_loop(0, _B * 128, zero, 0)

    iota = lax.iota(jnp.int32, _L)

    def start_fetch(c, wref, sem):
        base = basev[pl.ds(c, _L)][0] * 8
        for b in range(_B):
            pltpu.async_copy(
                sd_ref.at[pl.ds(b * (_C * _T) + c * _T + base, _W)],
                wref.at[pl.ds(b * _W, _W)], sem)

    def drain_fetch(wref, sem):
        for b in range(_B):
            pltpu.make_async_copy(
                sd_ref.at[pl.ds(0, _W)], wref.at[pl.ds(b * _W, _W)], sem
            ).wait()

    def compute(c, wref):
        base = basev[pl.ds(c, _L)][0] * 8
        x = xs_v[pl.ds(c, _L)][0]
        y = ys_v[pl.ds(c, _L)][0]
        ysp = jnp.full((_L,), y, jnp.int32)
        bsp = jnp.full((_L,), base, jnp.int32)
        bv0 = jnp.abs(ysp - (iota + by))
        bv1 = jnp.abs(ysp - (iota + (by + _L)))

        @plsc.parallel_loop(0, _BX, 1, unroll=8)
        def row_loop(lx):
            sa = jnp.abs(x - (bx + lx)) * 256
            for half, bv in ((0, bv0), (1, bv1)):
                tv = plsc.load_gather(t2d_v, [bv + sa])
                rel = tv - bsp
                for b in range(_B):
                    g = plsc.load_gather(wref, [rel + b * _W if b else rel])
                    plsc.addupdate(acc.at[b, lx, pl.ds(half * _L, _L)], g)

    start_fetch(0, win0, sem0)

    def pair_loop(cc, carry):
        c0 = 2 * cc
        drain_fetch(win0, sem0)
        start_fetch(c0 + 1, win1, sem1)
        compute(c0, win0)
        drain_fetch(win1, sem1)

        @pl.when(cc < _C // 2 - 1)
        def _():
            start_fetch(c0 + 2, win0, sem0)

        compute(c0 + 1, win1)
        return carry
    lax.fori_loop(0, _C // 2, pair_loop, 0)

    for b in range(_B):
        pltpu.sync_copy(acc.at[b], out_ref.at[b, pl.ds(bx, _BX), pl.ds(by, _BY)])


def _norm_body(x_ref, o_ref):
    v = x_ref[...]
    mn = jnp.min(v)
    mx = jnp.max(v)
    o_ref[...] = (v - mn) / (mx - mn)


def _normalize(img):
    return pl.pallas_call(
        _norm_body,
        grid=(_B,),
        in_specs=[pl.BlockSpec((1, _NX, _NY), lambda b: (b, 0, 0))],
        out_specs=pl.BlockSpec((1, _NX, _NY), lambda b: (b, 0, 0)),
        out_shape=jax.ShapeDtypeStruct((_B, _NX, _NY), jnp.float32),
    )(img)


def kernel(sensor_data, sensor_mask):
    sd_flat = sensor_data.reshape(_B * _C * _T)
    t2d = _t2d_table().reshape(_NX * _NY)
    xs = jnp.pad(sensor_mask[:, 0], (0, _L))
    ys = jnp.pad(sensor_mask[:, 1], (0, _L))
    raw = _sc_das(sd_flat, t2d, xs, ys)
    return _normalize(raw)


# speedup vs baseline: 1.0333x; 1.0333x over previous
import functools

import jax
import jax.numpy as jnp
from jax import lax
from jax.experimental import pallas as pl
from jax.experimental.pallas import tpu as pltpu
from jax.experimental.pallas import tpu_sc as plsc

_B, _C, _T = 8, 128, 5120
_NX, _NY = 256, 256
_DX, _DY = 0.001, 0.001
_VS, _DT = 1550.0, 5e-08
_TCLIP = 4655
_W = 1024
_NC, _NS = 2, 16
_BX, _BY = 64, 32
_L = 16


def _t2d_body(o_ref):
    a = lax.broadcasted_iota(jnp.int32, (_NX, _NY), 0).astype(jnp.float32)
    b = lax.broadcasted_iota(jnp.int32, (_NX, _NY), 1).astype(jnp.float32)
    dis = jnp.sqrt((a * _DX) ** 2 + (b * _DY) ** 2)
    t = (dis / _VS / _DT).astype(jnp.int32)
    o_ref[...] = jnp.clip(t, 0, _TCLIP)


def _t2d_table():
    return pl.pallas_call(
        _t2d_body,
        out_shape=jax.ShapeDtypeStruct((_NX, _NY), jnp.int32),
    )()


_mesh = plsc.VectorSubcoreMesh(core_axis_name="core", subcore_axis_name="sub")


@functools.partial(
    pl.kernel,
    out_type=jax.ShapeDtypeStruct((_B, _NX, _NY), jnp.float32),
    mesh=_mesh,
    compiler_params=pltpu.CompilerParams(
        use_tc_tiling_on_sc=False, needs_layout_passes=False),
    scratch_types=[
        pltpu.VMEM((_NX * _NY,), jnp.int32),
        pltpu.VMEM((_C + _L,), jnp.int32),
        pltpu.VMEM((_C + _L,), jnp.int32),
        pltpu.VMEM((_C + _L,), jnp.int32),
        pltpu.VMEM((_W * _B,), jnp.float32),
        pltpu.VMEM((_W * _B,), jnp.float32),
        pltpu.VMEM((_B, _BX, _BY), jnp.float32),
        pltpu.SemaphoreType.DMA,
        pltpu.SemaphoreType.DMA,
    ],
)
def _sc_das(sd_ref, t2d_ref, xs_ref, ys_ref, out_ref,
            t2d_v, xs_v, ys_v, basev, win0, win1, acc, sem0, sem1):
    cid = lax.axis_index("core")
    sid = lax.axis_index("sub")
    wid = sid * _NC + cid
    bx = (wid // 8) * _BX
    by = (wid % 8) * _BY

    pltpu.sync_copy(t2d_ref, t2d_v)
    pltpu.sync_copy(xs_ref, xs_v)
    pltpu.sync_copy(ys_ref, ys_v)

    def c_base(g, carry):
        xv = xs_v[pl.ds(g * _L, _L)]
        yv = ys_v[pl.ds(g * _L, _L)]
        amin = jnp.abs(jnp.clip(xv, bx, bx + _BX - 1) - xv)
        bmin = jnp.abs(jnp.clip(yv, by, by + _BY - 1) - yv)
        tmin = plsc.load_gather(t2d_v, [amin * 256 + bmin])
        basev[pl.ds(g * _L, _L)] = jnp.minimum(tmin >> 3, (_T - _W) // 8)
        return carry
    lax.fori_loop(0, _C // _L, c_base, 0)

    z = jnp.zeros((_L,), jnp.float32)
    def zero(k, carry):
        acc[k >> 7, (k >> 1) & 63, pl.ds((k & 1) * _L, _L)] = z
        return carry
    lax.fori_loop(0, _B * 128, zero, 0)

    iota = lax.iota(jnp.int32, _L)

    def start_fetch(c, wref, sem):
        base = basev[pl.ds(c, _L)][0] * 8
        for b in range(_B):
            pltpu.async_copy(
                sd_ref.at[pl.ds(b * (_C * _T) + c * _T + base, _W)],
                wref.at[pl.ds(b * _W, _W)], sem)

    def drain_fetch(wref, sem):
        for b in range(_B):
            pltpu.make_async_copy(
                sd_ref.at[pl.ds(0, _W)], wref.at[pl.ds(b * _W, _W)], sem
            ).wait()

    def compute(c, wref):
        base = basev[pl.ds(c, _L)][0] * 8
        x = xs_v[pl.ds(c, _L)][0]
        y = ys_v[pl.ds(c, _L)][0]
        ysp = jnp.full((_L,), y, jnp.int32)
        bsp = jnp.full((_L,), base, jnp.int32)
        bv0 = jnp.abs(ysp - (iota + by))
        bv1 = jnp.abs(ysp - (iota + (by + _L)))

        @plsc.parallel_loop(0, _BX, 1, unroll=2)
        def row_loop(lx):
            sa = jnp.abs(x - (bx + lx)) * 256
            for half, bv in ((0, bv0), (1, bv1)):
                tv = plsc.load_gather(t2d_v, [bv + sa])
                rel = tv - bsp
                for b in range(_B):
                    g = plsc.load_gather(wref, [rel + b * _W if b else rel])
                    plsc.addupdate(acc.at[b, lx, pl.ds(half * _L, _L)], g)

    start_fetch(0, win0, sem0)

    def pair_loop(cc, carry):
        c0 = 2 * cc
        drain_fetch(win0, sem0)
        start_fetch(c0 + 1, win1, sem1)
        compute(c0, win0)
        drain_fetch(win1, sem1)

        @pl.when(cc < _C // 2 - 1)
        def _():
            start_fetch(c0 + 2, win0, sem0)

        compute(c0 + 1, win1)
        return carry
    lax.fori_loop(0, _C // 2, pair_loop, 0)

    for b in range(_B):
        pltpu.sync_copy(acc.at[b], out_ref.at[b, pl.ds(bx, _BX), pl.ds(by, _BY)])


def _norm_body(x_ref, o_ref):
    v = x_ref[...]
    mn = jnp.min(v)
    mx = jnp.max(v)
    o_ref[...] = (v - mn) / (mx - mn)


def _normalize(img):
    return pl.pallas_call(
        _norm_body,
        grid=(_B,),
        in_specs=[pl.BlockSpec((1, _NX, _NY), lambda b: (b, 0, 0))],
        out_specs=pl.BlockSpec((1, _NX, _NY), lambda b: (b, 0, 0)),
        out_shape=jax.ShapeDtypeStruct((_B, _NX, _NY), jnp.float32),
    )(img)


def kernel(sensor_data, sensor_mask):
    sd_flat = sensor_data.reshape(_B * _C * _T)
    t2d = _t2d_table().reshape(_NX * _NY)
    xs = jnp.pad(sensor_mask[:, 0], (0, _L))
    ys = jnp.pad(sensor_mask[:, 1], (0, _L))
    raw = _sc_das(sd_flat, t2d, xs, ys)
    return _normalize(raw)


# speedup vs baseline: 1.1606x; 1.1232x over previous
import functools

import jax
import jax.numpy as jnp
from jax import lax
from jax.experimental import pallas as pl
from jax.experimental.pallas import tpu as pltpu
from jax.experimental.pallas import tpu_sc as plsc

_B, _C, _T = 8, 128, 5120
_NX, _NY = 256, 256
_DX, _DY = 0.001, 0.001
_VS, _DT = 1550.0, 5e-08
_TCLIP = 4655
_W = 1024
_NC, _NS = 2, 16
_BX, _BY = 64, 32
_L = 16
_NCH = 4


def _t2d_body(o_ref):
    a = lax.broadcasted_iota(jnp.int32, (_NX, _NY // 2), 0).astype(jnp.float32)
    k = lax.broadcasted_iota(jnp.int32, (_NX, _NY // 2), 1)

    def tof(bcol):
        dis = jnp.sqrt((a * _DX) ** 2 + (bcol * _DY) ** 2)
        return jnp.clip((dis / _VS / _DT).astype(jnp.int32), 0, _TCLIP)

    te = tof((2 * k).astype(jnp.float32))
    to = tof((2 * k + 1).astype(jnp.float32))
    o_ref[...] = te | (to << 16)


def _t2d_table():
    return pl.pallas_call(
        _t2d_body,
        out_shape=jax.ShapeDtypeStruct((_NX, _NY // 2), jnp.int32),
    )()


_mesh = plsc.VectorSubcoreMesh(core_axis_name="core", subcore_axis_name="sub")


@functools.partial(
    pl.kernel,
    out_type=jax.ShapeDtypeStruct((_B, _NX, _NY), jnp.float32),
    mesh=_mesh,
    compiler_params=pltpu.CompilerParams(
        use_tc_tiling_on_sc=False, needs_layout_passes=False),
    scratch_types=[
        pltpu.VMEM((_NX, _NY // 2), jnp.int32),
        pltpu.VMEM((_C + _L,), jnp.int32),
        pltpu.VMEM((_C + _L,), jnp.int32),
        pltpu.VMEM((_C + _L,), jnp.int32),
        pltpu.VMEM((2, _NCH, _B, _W), jnp.float32),
        pltpu.VMEM((_B, _BX, _BY), jnp.float32),
        pltpu.SemaphoreType.DMA,
        pltpu.SemaphoreType.DMA,
    ],
)
def _sc_das(sd_ref, t2d_ref, xs_ref, ys_ref, out_ref,
            t2d_v, xs_v, ys_v, basev, win, acc, sem0, sem1):
    cid = lax.axis_index("core")
    sid = lax.axis_index("sub")
    wid = sid * _NC + cid
    bx = (wid // 8) * _BX
    by = (wid % 8) * _BY

    pltpu.sync_copy(t2d_ref, t2d_v)
    pltpu.sync_copy(xs_ref, xs_v)
    pltpu.sync_copy(ys_ref, ys_v)

    def c_base(g, carry):
        xv = xs_v[pl.ds(g * _L, _L)]
        yv = ys_v[pl.ds(g * _L, _L)]
        amin = jnp.abs(jnp.clip(xv, bx, bx + _BX - 1) - xv)
        bmin = jnp.abs(jnp.clip(yv, by, by + _BY - 1) - yv)
        word = plsc.load_gather(t2d_v, [amin, bmin >> 1])
        tmin = (word >> ((bmin & 1) << 4)) & 0xFFFF
        basev[pl.ds(g * _L, _L)] = jnp.minimum(tmin >> 3, (_T - _W) // 8)
        return carry
    lax.fori_loop(0, _C // _L, c_base, 0)

    z = jnp.zeros((_L,), jnp.float32)

    def zero(j, carry):
        acc[j >> 7, (j >> 1) & 63, pl.ds((j & 1) * _L, _L)] = z
        return carry
    lax.fori_loop(0, _B * 128, zero, 0)

    iota = lax.iota(jnp.int32, _L)

    def start_group_fetch(g, s, sem):
        for k in range(_NCH):
            c = g * _NCH + k
            base = basev[pl.ds(c, _L)][0] * 8
            for b in range(_B):
                pltpu.async_copy(
                    sd_ref.at[pl.ds(b * (_C * _T) + c * _T + base, _W)],
                    win.at[s, k, b], sem)

    def drain_group(s, sem):
        for k in range(_NCH):
            for b in range(_B):
                pltpu.make_async_copy(
                    sd_ref.at[pl.ds(0, _W)], win.at[s, k, b], sem).wait()

    def compute_group(g, s):
        prm = []
        for k in range(_NCH):
            c = g * _NCH + k
            base = basev[pl.ds(c, _L)][0] * 8
            x = xs_v[pl.ds(c, _L)][0]
            y = ys_v[pl.ds(c, _L)][0]
            ysp = jnp.full((_L,), y, jnp.int32)
            bsp = jnp.full((_L,), base, jnp.int32)
            bv0 = jnp.abs(ysp - (iota + by))
            bv1 = jnp.abs(ysp - (iota + (by + _L)))
            prm.append((x, bsp,
                        ((bv0 >> 1), ((bv0 & 1) << 4)),
                        ((bv1 >> 1), ((bv1 & 1) << 4))))

        @plsc.parallel_loop(0, _BX, 1, unroll=2)
        def row_loop(lx):
            for half in (0, 1):
                rels = []
                for k in range(_NCH):
                    x, bsp, h0, h1 = prm[k]
                    bvh, sh = h0 if half == 0 else h1
                    av = jnp.full((_L,), jnp.abs(x - (bx + lx)), jnp.int32)
                    word = plsc.load_gather(t2d_v, [av, bvh])
                    rels.append(((word >> sh) & 0xFFFF) - bsp)
                for b in range(_B):
                    tot = None
                    for k in range(_NCH):
                        gat = plsc.load_gather(win.at[s, k, b], [rels[k]])
                        tot = gat if tot is None else tot + gat
                    plsc.addupdate(acc.at[b, lx, pl.ds(half * _L, _L)], tot)

    start_group_fetch(0, 0, sem0)
    ngroups = _C // _NCH

    def gpair_loop(gg, carry):
        g0 = 2 * gg
        drain_group(0, sem0)
        start_group_fetch(g0 + 1, 1, sem1)
        compute_group(g0, 0)
        drain_group(1, sem1)

        @pl.when(gg < ngroups // 2 - 1)
        def _():
            start_group_fetch(g0 + 2, 0, sem0)

        compute_group(g0 + 1, 1)
        return carry
    lax.fori_loop(0, ngroups // 2, gpair_loop, 0)

    for b in range(_B):
        pltpu.sync_copy(acc.at[b], out_ref.at[b, pl.ds(bx, _BX), pl.ds(by, _BY)])


def _norm_body(x_ref, o_ref):
    v = x_ref[...]
    mn = jnp.min(v)
    mx = jnp.max(v)
    o_ref[...] = (v - mn) / (mx - mn)


def _normalize(img):
    return pl.pallas_call(
        _norm_body,
        grid=(_B,),
        in_specs=[pl.BlockSpec((1, _NX, _NY), lambda b: (b, 0, 0))],
        out_specs=pl.BlockSpec((1, _NX, _NY), lambda b: (b, 0, 0)),
        out_shape=jax.ShapeDtypeStruct((_B, _NX, _NY), jnp.float32),
    )(img)


def kernel(sensor_data, sensor_mask):
    sd_flat = sensor_data.reshape(_B * _C * _T)
    t2d = _t2d_table()
    xs = jnp.pad(sensor_mask[:, 0], (0, _L))
    ys = jnp.pad(sensor_mask[:, 1], (0, _L))
    raw = _sc_das(sd_flat, t2d, xs, ys)
    return _normalize(raw)
